# baseline (speedup 1.0000x reference)
import functools

import jax
import jax.numpy as jnp
from jax import lax
from jax.experimental import pallas as pl
from jax.experimental.pallas import tpu as pltpu
from jax.experimental.pallas import tpu_sc as plsc

N = 50000
E = 800000
B = 64
D_IN = 25
H = 64
D_OUT = 128

NS = 16
CH = 128
NSL = 6
SW = 3
IR = NSL * SW
NWAVE = 22
NCH = IR * NWAVE
EPT = NCH * CH
E_PAD = EPT * NS
NWAVE1 = 11
NCH1 = IR * NWAVE1
ROWS = 50048
RPT = ROWS // NS
RPT_LAST = N - (NS - 1) * RPT

R = 5000
W1 = 32


def _sc_pipeline(xh, src, dst, zg, acc, sidx, didx, rows, gsem, ssem,
                 base0, nwave):
    def drain(j):
        pltpu.make_async_copy(zg, rows.at[j], ssem.at[j]).wait()

    def wave(w, carry):
        rowb = base0 + w * IR

        @pl.when(w > 0)
        def _():
            for j in range(NSL):
                drain(j)

        pltpu.sync_copy(src.at[pl.ds(rowb, IR)], sidx)
        pltpu.sync_copy(dst.at[pl.ds(rowb, IR)], didx)
        for sw in range(SW):
            if sw > 0:
                for j in range(NSL):
                    drain(j)
            gd = [pltpu.async_copy(xh.at[sidx.at[sw * NSL + j]],
                                   rows.at[j], gsem.at[j])
                  for j in range(NSL)]
            for j in range(NSL):
                gd[j].wait()
                pltpu.async_copy(rows.at[j], acc.at[didx.at[sw * NSL + j]],
                                 ssem.at[j], add=True)
        return carry

    lax.fori_loop(0, nwave, wave, 0)
    for j in range(NSL):
        drain(j)


def _copy_out(s, acc, out):
    r0 = s * RPT

    @pl.when(s < NS - 1)
    def _():
        pltpu.sync_copy(acc.at[pl.ds(r0, RPT)], out.at[pl.ds(r0, RPT)])

    @pl.when(s == NS - 1)
    def _():
        pltpu.sync_copy(acc.at[pl.ds(r0, RPT_LAST)],
                        out.at[pl.ds(r0, RPT_LAST)])


def _sc_scratch(Wh):
    return [
        pltpu.VMEM((IR, CH), jnp.int32),
        pltpu.VMEM((IR, CH), jnp.int32),
        pltpu.VMEM((NSL, CH, Wh), jnp.float32),
        pltpu.VMEM_SHARED((ROWS, Wh), jnp.float32),
        pltpu.SemaphoreType.DMA((NSL,)),
        pltpu.SemaphoreType.DMA((NSL,)),
    ]


@functools.cache
def _make_sc_agg_es():
    mesh = plsc.VectorSubcoreMesh(core_axis_name="c", subcore_axis_name="s")

    def body(x, src, dst, zrows, zg, out0, out1, sidx, didx, rows, acc,
             gsem, ssem):
        c = lax.axis_index("c")
        s = lax.axis_index("s")
        pltpu.sync_copy(zrows, acc.at[pl.ds(s * RPT, RPT)])
        plsc.subcore_barrier()
        base0 = (c * NS + s) * NCH1
        _sc_pipeline(x, src, dst, zg, acc, sidx, didx, rows, gsem, ssem,
                     base0, NWAVE1)
        plsc.subcore_barrier()

        @pl.when(c == 0)
        def _():
            _copy_out(s, acc, out0)

        @pl.when(c == 1)
        def _():
            _copy_out(s, acc, out1)

    return pl.kernel(
        body,
        out_type=(jax.ShapeDtypeStruct((N, W1), jnp.float32),
                  jax.ShapeDtypeStruct((N, W1), jnp.float32)),
        mesh=mesh,
        scratch_types=_sc_scratch(W1),
        compiler_params=pltpu.CompilerParams(use_tc_tiling_on_sc=False),
        name="sc_agg_es",
    )


@functools.cache
def _make_sc_agg_fs(Wh):
    mesh = plsc.VectorSubcoreMesh(core_axis_name="c", subcore_axis_name="s")

    def body(xlo, xhi, src, dst, zrows, zg, outlo, outhi, sidx, didx, rows,
             acc, gsem, ssem):
        c = lax.axis_index("c")
        s = lax.axis_index("s")
        pltpu.sync_copy(zrows, acc.at[pl.ds(s * RPT, RPT)])
        plsc.subcore_barrier()
        base0 = s * NCH

        def run(xh, out):
            _sc_pipeline(xh, src, dst, zg, acc, sidx, didx, rows, gsem,
                         ssem, base0, NWAVE)
            plsc.subcore_barrier()
            _copy_out(s, acc, out)

        @pl.when(c == 0)
        def _():
            run(xlo, outlo)

        @pl.when(c == 1)
        def _():
            run(xhi, outhi)

    return pl.kernel(
        body,
        out_type=(jax.ShapeDtypeStruct((N, Wh), jnp.float32),
                  jax.ShapeDtypeStruct((N, Wh), jnp.float32)),
        mesh=mesh,
        scratch_types=_sc_scratch(Wh),
        compiler_params=pltpu.CompilerParams(use_tc_tiling_on_sc=False),
        name=f"sc_agg_w{Wh}",
    )


def _pool_update(p, xr, bat, i):
    @pl.when(i == 0)
    def _():
        p[...] = jnp.full((B, H), -jnp.inf, jnp.float32)

    lo = bat[0, 0]
    hi = bat[R - 1, 0]

    def seg(bseg, carry):
        m = jnp.max(jnp.where(bat == bseg, xr, -jnp.inf),
                    axis=0, keepdims=True)
        p[pl.ds(bseg, 1), :] = jnp.maximum(p[pl.ds(bseg, 1), :], m)
        return carry

    lax.fori_loop(lo, hi + 1, seg, 0)


def _make_tc_layer1():
    grid = N // R

    def body(a0, a1, xf, bat, wl, wr, bias,
             xnlo, xnhi, p, invd_out):
        i = pl.program_id(0)
        a = a0[...] + a1[...]
        invd = 1.0 / jnp.maximum(a[:, D_IN:D_IN + 1], 1.0)
        invd_out[...] = invd
        h = ((a * invd) @ wl[...]
             + xf[...] @ wr[...]
             + bias[...])
        xr = jnp.maximum(h, 0.0)
        xnlo[...] = xr[:, :H // 2]
        xnhi[...] = xr[:, H // 2:]
        _pool_update(p, xr, bat[...], i)

    return pl.pallas_call(
        body,
        grid=(grid,),
        in_specs=[
            pl.BlockSpec((R, W1), lambda i: (i, 0)),
            pl.BlockSpec((R, W1), lambda i: (i, 0)),
            pl.BlockSpec((R, W1), lambda i: (i, 0)),
            pl.BlockSpec((R, 1), lambda i: (i, 0)),
            pl.BlockSpec((W1, H), lambda i: (0, 0)),
            pl.BlockSpec((W1, H), lambda i: (0, 0)),
            pl.BlockSpec((1, H), lambda i: (0, 0)),
        ],
        out_specs=[
            pl.BlockSpec((R, H // 2), lambda i: (i, 0)),
            pl.BlockSpec((R, H // 2), lambda i: (i, 0)),
            pl.BlockSpec((B, H), lambda i: (0, 0)),
            pl.BlockSpec((R, 1), lambda i: (i, 0)),
        ],
        out_shape=[
            jax.ShapeDtypeStruct((N, H // 2), jnp.float32),
            jax.ShapeDtypeStruct((N, H // 2), jnp.float32),
            jax.ShapeDtypeStruct((B, H), jnp.float32),
            jax.ShapeDtypeStruct((N, 1), jnp.float32),
        ],
        name="tc_layer1",
    )


def _make_tc_layer(last):
    grid = N // R
    Wa = H // 2

    def body(*refs):
        (alo, ahi, xlo, xhi, invd_r, bat, wllo, wlhi, wrlo, wrhi, bias,
         *rest) = refs
        if last:
            p1, p2, a1m, a2m, a3m, b1m, w2m, b2m, p, outm = rest
        else:
            xnlo, xnhi, p = rest
        i = pl.program_id(0)
        invd = invd_r[...]
        h = ((alo[...] * invd) @ wllo[...]
             + (ahi[...] * invd) @ wlhi[...]
             + xlo[...] @ wrlo[...]
             + xhi[...] @ wrhi[...]
             + bias[...])
        xr = jnp.maximum(h, 0.0)
        if not last:
            xnlo[...] = xr[:, :H // 2]
            xnhi[...] = xr[:, H // 2:]
        _pool_update(p, xr, bat[...], i)
        if last:
            @pl.when(i == grid - 1)
            def _():
                hm = jnp.maximum(p1[...] @ a1m[...] + p2[...] @ a2m[...]
                                 + p[...] @ a3m[...] + b1m[...], 0.0)
                outm[...] = hm @ w2m[...] + b2m[...]

    in_specs = [
        pl.BlockSpec((R, Wa), lambda i: (i, 0)),
        pl.BlockSpec((R, Wa), lambda i: (i, 0)),
        pl.BlockSpec((R, Wa), lambda i: (i, 0)),
        pl.BlockSpec((R, Wa), lambda i: (i, 0)),
        pl.BlockSpec((R, 1), lambda i: (i, 0)),
        pl.BlockSpec((R, 1), lambda i: (i, 0)),
        pl.BlockSpec((Wa, H), lambda i: (0, 0)),
        pl.BlockSpec((Wa, H), lambda i: (0, 0)),
        pl.BlockSpec((Wa, H), lambda i: (0, 0)),
        pl.BlockSpec((Wa, H), lambda i: (0, 0)),
        pl.BlockSpec((1, H), lambda i: (0, 0)),
    ]
    out_specs = [pl.BlockSpec((B, H), lambda i: (0, 0))]
    out_shape = [jax.ShapeDtypeStruct((B, H), jnp.float32)]
    if last:
        in_specs += [
            pl.BlockSpec((B, H), lambda i: (0, 0)),
            pl.BlockSpec((B, H), lambda i: (0, 0)),
            pl.BlockSpec((H, H), lambda i: (0, 0)),
            pl.BlockSpec((H, H), lambda i: (0, 0)),
            pl.BlockSpec((H, H), lambda i: (0, 0)),
            pl.BlockSpec((1, H), lambda i: (0, 0)),
            pl.BlockSpec((H, D_OUT), lambda i: (0, 0)),
            pl.BlockSpec((1, D_OUT), lambda i: (0, 0)),
        ]
        out_specs.append(pl.BlockSpec((B, D_OUT), lambda i: (0, 0)))
        out_shape.append(jax.ShapeDtypeStruct((B, D_OUT), jnp.float32))
    else:
        out_specs = [
            pl.BlockSpec((R, H // 2), lambda i: (i, 0)),
            pl.BlockSpec((R, H // 2), lambda i: (i, 0)),
        ] + out_specs
        out_shape = [
            jax.ShapeDtypeStruct((N, H // 2), jnp.float32),
            jax.ShapeDtypeStruct((N, H // 2), jnp.float32),
        ] + out_shape

    return pl.pallas_call(
        body,
        grid=(grid,),
        in_specs=in_specs,
        out_specs=out_specs,
        out_shape=out_shape,
        name="tc_layer3" if last else "tc_layer2",
    )


_tc_layer1 = _make_tc_layer1()
_tc_layer2 = _make_tc_layer(last=False)
_tc_layer3 = _make_tc_layer(last=True)


def kernel(x, edge_index, batch, W_l1, W_r1, b1, W_l2, W_r2, b2,
           W_l3, W_r3, b3, W_lin1, b_lin1, W_lin2, b_lin2):
    f32 = jnp.float32
    src = edge_index[0]
    dst = edge_index[1]
    pad = E_PAD - E
    src_p = jnp.concatenate([src, jnp.zeros((pad,), jnp.int32)])
    src_p = src_p.reshape(E_PAD // CH, CH)
    dst_p = jnp.concatenate([dst, jnp.full((pad,), N, jnp.int32)])
    dst_p = dst_p.reshape(E_PAD // CH, CH)

    x_pad = jnp.concatenate(
        [x, jnp.ones((N, 1), f32), jnp.zeros((N, W1 - D_IN - 1), f32)],
        axis=1)
    z32 = jnp.zeros((RPT, 32), f32)
    zg32 = jnp.zeros((CH, 32), f32)
    batch2 = batch[:, None]

    def pad_t(w):
        wt = w.T
        return jnp.concatenate([wt, jnp.zeros((W1 - D_IN, H), f32)], axis=0)

    wl1 = pad_t(W_l1)
    wr1 = pad_t(W_r1)
    wl2 = W_l2.T
    wr2 = W_r2.T
    wl3 = W_l3.T
    wr3 = W_r3.T
    w1t = W_lin1.T

    a10, a11 = _make_sc_agg_es()(x_pad, src_p, dst_p, z32, zg32)
    x1_lo, x1_hi, p1, invd = _tc_layer1(
        a10, a11, x_pad, batch2, wl1, wr1, b1[None])

    a2_lo, a2_hi = _make_sc_agg_fs(32)(x1_lo, x1_hi, src_p, dst_p, z32, zg32)
    x2_lo, x2_hi, p2 = _tc_layer2(
        a2_lo, a2_hi, x1_lo, x1_hi, invd, batch2,
        wl2[:32], wl2[32:], wr2[:32], wr2[32:], b2[None])

    a3_lo, a3_hi = _make_sc_agg_fs(32)(x2_lo, x2_hi, src_p, dst_p, z32, zg32)
    _, out = _tc_layer3(
        a3_lo, a3_hi, x2_lo, x2_hi, invd, batch2,
        wl3[:32], wl3[32:], wr3[:32], wr3[32:], b3[None],
        p1, p2, w1t[:H], w1t[H:2 * H], w1t[2 * H:],
        b_lin1[None], W_lin2.T, b_lin2[None])
    return out


# speedup vs baseline: 1.0431x; 1.0431x over previous
import functools

import jax
import jax.numpy as jnp
from jax import lax
from jax.experimental import pallas as pl
from jax.experimental.pallas import tpu as pltpu
from jax.experimental.pallas import tpu_sc as plsc

N = 50000
E = 800000
B = 64
D_IN = 25
H = 64
D_OUT = 128

NS = 16
CH = 128
NSL = 6
SW = 3
IR = NSL * SW
NWAVE = 22
NCH = IR * NWAVE
EPT = NCH * CH
E_PAD = EPT * NS
NWAVE1 = 11
NCH1 = IR * NWAVE1
ROWS = 50048
RPT = ROWS // NS
RPT_LAST = N - (NS - 1) * RPT

R = 2000
W1 = 32


def _sc_pipeline(xh, src, dst, zg, acc, sidx, didx, rows, gsem, ssem,
                 base0, nwave):
    def drain(j):
        pltpu.make_async_copy(zg, rows.at[j], ssem.at[j]).wait()

    def wave(w, carry):
        rowb = base0 + w * IR

        @pl.when(w > 0)
        def _():
            for j in range(NSL):
                drain(j)

        pltpu.sync_copy(src.at[pl.ds(rowb, IR)], sidx)
        pltpu.sync_copy(dst.at[pl.ds(rowb, IR)], didx)
        for sw in range(SW):
            if sw > 0:
                for j in range(NSL):
                    drain(j)
            gd = [pltpu.async_copy(xh.at[sidx.at[sw * NSL + j]],
                                   rows.at[j], gsem.at[j])
                  for j in range(NSL)]
            for j in range(NSL):
                gd[j].wait()
                pltpu.async_copy(rows.at[j], acc.at[didx.at[sw * NSL + j]],
                                 ssem.at[j], add=True)
        return carry

    lax.fori_loop(0, nwave, wave, 0)
    for j in range(NSL):
        drain(j)


def _copy_out(s, acc, out):
    r0 = s * RPT

    @pl.when(s < NS - 1)
    def _():
        pltpu.sync_copy(acc.at[pl.ds(r0, RPT)], out.at[pl.ds(r0, RPT)])

    @pl.when(s == NS - 1)
    def _():
        pltpu.sync_copy(acc.at[pl.ds(r0, RPT_LAST)],
                        out.at[pl.ds(r0, RPT_LAST)])


def _sc_scratch(Wh):
    return [
        pltpu.VMEM((IR, CH), jnp.int32),
        pltpu.VMEM((IR, CH), jnp.int32),
        pltpu.VMEM((NSL, CH, Wh), jnp.float32),
        pltpu.VMEM_SHARED((ROWS, Wh), jnp.float32),
        pltpu.SemaphoreType.DMA((NSL,)),
        pltpu.SemaphoreType.DMA((NSL,)),
    ]


@functools.cache
def _make_sc_agg_es():
    mesh = plsc.VectorSubcoreMesh(core_axis_name="c", subcore_axis_name="s")

    def body(x, src, dst, zrows, zg, out0, out1, sidx, didx, rows, acc,
             gsem, ssem):
        c = lax.axis_index("c")
        s = lax.axis_index("s")
        pltpu.sync_copy(zrows, acc.at[pl.ds(s * RPT, RPT)])
        plsc.subcore_barrier()
        base0 = (c * NS + s) * NCH1
        _sc_pipeline(x, src, dst, zg, acc, sidx, didx, rows, gsem, ssem,
                     base0, NWAVE1)
        plsc.subcore_barrier()

        @pl.when(c == 0)
        def _():
            _copy_out(s, acc, out0)

        @pl.when(c == 1)
        def _():
            _copy_out(s, acc, out1)

    return pl.kernel(
        body,
        out_type=(jax.ShapeDtypeStruct((N, W1), jnp.float32),
                  jax.ShapeDtypeStruct((N, W1), jnp.float32)),
        mesh=mesh,
        scratch_types=_sc_scratch(W1),
        compiler_params=pltpu.CompilerParams(use_tc_tiling_on_sc=False),
        name="sc_agg_es",
    )


@functools.cache
def _make_sc_agg_fs(Wh):
    mesh = plsc.VectorSubcoreMesh(core_axis_name="c", subcore_axis_name="s")

    def body(xlo, xhi, src, dst, zrows, zg, outlo, outhi, sidx, didx, rows,
             acc, gsem, ssem):
        c = lax.axis_index("c")
        s = lax.axis_index("s")
        pltpu.sync_copy(zrows, acc.at[pl.ds(s * RPT, RPT)])
        plsc.subcore_barrier()
        base0 = s * NCH

        def run(xh, out):
            _sc_pipeline(xh, src, dst, zg, acc, sidx, didx, rows, gsem,
                         ssem, base0, NWAVE)
            plsc.subcore_barrier()
            _copy_out(s, acc, out)

        @pl.when(c == 0)
        def _():
            run(xlo, outlo)

        @pl.when(c == 1)
        def _():
            run(xhi, outhi)

    return pl.kernel(
        body,
        out_type=(jax.ShapeDtypeStruct((N, Wh), jnp.float32),
                  jax.ShapeDtypeStruct((N, Wh), jnp.float32)),
        mesh=mesh,
        scratch_types=_sc_scratch(Wh),
        compiler_params=pltpu.CompilerParams(use_tc_tiling_on_sc=False),
        name=f"sc_agg_w{Wh}",
    )


def _pool_update(p, xr, bat, i):
    @pl.when(i == 0)
    def _():
        p[...] = jnp.full((B, H), -jnp.inf, jnp.float32)

    lo = bat[0, 0]
    hi = bat[R - 1, 0]

    def seg(bseg, carry):
        m = jnp.max(jnp.where(bat == bseg, xr, -jnp.inf),
                    axis=0, keepdims=True)
        p[pl.ds(bseg, 1), :] = jnp.maximum(p[pl.ds(bseg, 1), :], m)
        return carry

    lax.fori_loop(lo, hi + 1, seg, 0)


def _make_tc_layer1():
    grid = N // R

    def body(a0, a1, xf, bat, wl, wr, bias,
             xnlo, xnhi, p, invd_out):
        i = pl.program_id(0)
        a = a0[...] + a1[...]
        invd = 1.0 / jnp.maximum(a[:, D_IN:D_IN + 1], 1.0)
        invd_out[...] = invd
        h = ((a * invd) @ wl[...]
             + xf[...] @ wr[...]
             + bias[...])
        xr = jnp.maximum(h, 0.0)
        xnlo[...] = xr[:, :H // 2]
        xnhi[...] = xr[:, H // 2:]
        _pool_update(p, xr, bat[...], i)

    return pl.pallas_call(
        body,
        grid=(grid,),
        in_specs=[
            pl.BlockSpec((R, W1), lambda i: (i, 0)),
            pl.BlockSpec((R, W1), lambda i: (i, 0)),
            pl.BlockSpec((R, W1), lambda i: (i, 0)),
            pl.BlockSpec((R, 1), lambda i: (i, 0)),
            pl.BlockSpec((W1, H), lambda i: (0, 0)),
            pl.BlockSpec((W1, H), lambda i: (0, 0)),
            pl.BlockSpec((1, H), lambda i: (0, 0)),
        ],
        out_specs=[
            pl.BlockSpec((R, H // 2), lambda i: (i, 0)),
            pl.BlockSpec((R, H // 2), lambda i: (i, 0)),
            pl.BlockSpec((B, H), lambda i: (0, 0)),
            pl.BlockSpec((R, 1), lambda i: (i, 0)),
        ],
        out_shape=[
            jax.ShapeDtypeStruct((N, H // 2), jnp.float32),
            jax.ShapeDtypeStruct((N, H // 2), jnp.float32),
            jax.ShapeDtypeStruct((B, H), jnp.float32),
            jax.ShapeDtypeStruct((N, 1), jnp.float32),
        ],
        name="tc_layer1",
    )


def _make_tc_layer(last):
    grid = N // R
    Wa = H // 2

    def body(*refs):
        (alo, ahi, xlo, xhi, invd_r, bat, wllo, wlhi, wrlo, wrhi, bias,
         *rest) = refs
        if last:
            p1, p2, a1m, a2m, a3m, b1m, w2m, b2m, p, outm = rest
        else:
            xnlo, xnhi, p = rest
        i = pl.program_id(0)
        invd = invd_r[...]
        h = ((alo[...] * invd) @ wllo[...]
             + (ahi[...] * invd) @ wlhi[...]
             + xlo[...] @ wrlo[...]
             + xhi[...] @ wrhi[...]
             + bias[...])
        xr = jnp.maximum(h, 0.0)
        if not last:
            xnlo[...] = xr[:, :H // 2]
            xnhi[...] = xr[:, H // 2:]
        _pool_update(p, xr, bat[...], i)
        if last:
            @pl.when(i == grid - 1)
            def _():
                hm = jnp.maximum(p1[...] @ a1m[...] + p2[...] @ a2m[...]
                                 + p[...] @ a3m[...] + b1m[...], 0.0)
                outm[...] = hm @ w2m[...] + b2m[...]

    in_specs = [
        pl.BlockSpec((R, Wa), lambda i: (i, 0)),
        pl.BlockSpec((R, Wa), lambda i: (i, 0)),
        pl.BlockSpec((R, Wa), lambda i: (i, 0)),
        pl.BlockSpec((R, Wa), lambda i: (i, 0)),
        pl.BlockSpec((R, 1), lambda i: (i, 0)),
        pl.BlockSpec((R, 1), lambda i: (i, 0)),
        pl.BlockSpec((Wa, H), lambda i: (0, 0)),
        pl.BlockSpec((Wa, H), lambda i: (0, 0)),
        pl.BlockSpec((Wa, H), lambda i: (0, 0)),
        pl.BlockSpec((Wa, H), lambda i: (0, 0)),
        pl.BlockSpec((1, H), lambda i: (0, 0)),
    ]
    out_specs = [pl.BlockSpec((B, H), lambda i: (0, 0))]
    out_shape = [jax.ShapeDtypeStruct((B, H), jnp.float32)]
    if last:
        in_specs += [
            pl.BlockSpec((B, H), lambda i: (0, 0)),
            pl.BlockSpec((B, H), lambda i: (0, 0)),
            pl.BlockSpec((H, H), lambda i: (0, 0)),
            pl.BlockSpec((H, H), lambda i: (0, 0)),
            pl.BlockSpec((H, H), lambda i: (0, 0)),
            pl.BlockSpec((1, H), lambda i: (0, 0)),
            pl.BlockSpec((H, D_OUT), lambda i: (0, 0)),
            pl.BlockSpec((1, D_OUT), lambda i: (0, 0)),
        ]
        out_specs.append(pl.BlockSpec((B, D_OUT), lambda i: (0, 0)))
        out_shape.append(jax.ShapeDtypeStruct((B, D_OUT), jnp.float32))
    else:
        out_specs = [
            pl.BlockSpec((R, H // 2), lambda i: (i, 0)),
            pl.BlockSpec((R, H // 2), lambda i: (i, 0)),
        ] + out_specs
        out_shape = [
            jax.ShapeDtypeStruct((N, H // 2), jnp.float32),
            jax.ShapeDtypeStruct((N, H // 2), jnp.float32),
        ] + out_shape

    return pl.pallas_call(
        body,
        grid=(grid,),
        in_specs=in_specs,
        out_specs=out_specs,
        out_shape=out_shape,
        name="tc_layer3" if last else "tc_layer2",
    )


_tc_layer1 = _make_tc_layer1()
_tc_layer2 = _make_tc_layer(last=False)
_tc_layer3 = _make_tc_layer(last=True)


def kernel(x, edge_index, batch, W_l1, W_r1, b1, W_l2, W_r2, b2,
           W_l3, W_r3, b3, W_lin1, b_lin1, W_lin2, b_lin2):
    f32 = jnp.float32
    src = edge_index[0]
    dst = edge_index[1]
    pad = E_PAD - E
    src_p = jnp.concatenate([src, jnp.zeros((pad,), jnp.int32)])
    src_p = src_p.reshape(E_PAD // CH, CH)
    dst_p = jnp.concatenate([dst, jnp.full((pad,), N, jnp.int32)])
    dst_p = dst_p.reshape(E_PAD // CH, CH)

    x_pad = jnp.concatenate(
        [x, jnp.ones((N, 1), f32), jnp.zeros((N, W1 - D_IN - 1), f32)],
        axis=1)
    z32 = jnp.zeros((RPT, 32), f32)
    zg32 = jnp.zeros((CH, 32), f32)
    batch2 = batch[:, None]

    def pad_t(w):
        wt = w.T
        return jnp.concatenate([wt, jnp.zeros((W1 - D_IN, H), f32)], axis=0)

    wl1 = pad_t(W_l1)
    wr1 = pad_t(W_r1)
    wl2 = W_l2.T
    wr2 = W_r2.T
    wl3 = W_l3.T
    wr3 = W_r3.T
    w1t = W_lin1.T

    a10, a11 = _make_sc_agg_es()(x_pad, src_p, dst_p, z32, zg32)
    x1_lo, x1_hi, p1, invd = _tc_layer1(
        a10, a11, x_pad, batch2, wl1, wr1, b1[None])

    a2_lo, a2_hi = _make_sc_agg_fs(32)(x1_lo, x1_hi, src_p, dst_p, z32, zg32)
    x2_lo, x2_hi, p2 = _tc_layer2(
        a2_lo, a2_hi, x1_lo, x1_hi, invd, batch2,
        wl2[:32], wl2[32:], wr2[:32], wr2[32:], b2[None])

    a3_lo, a3_hi = _make_sc_agg_fs(32)(x2_lo, x2_hi, src_p, dst_p, z32, zg32)
    _, out = _tc_layer3(
        a3_lo, a3_hi, x2_lo, x2_hi, invd, batch2,
        wl3[:32], wl3[32:], wr3[:32], wr3[32:], b3[None],
        p1, p2, w1t[:H], w1t[H:2 * H], w1t[2 * H:],
        b_lin1[None], W_lin2.T, b_lin2[None])
    return out


# speedup vs baseline: 1.3203x; 1.2658x over previous
import functools

import jax
import jax.numpy as jnp
from jax import lax
from jax.experimental import pallas as pl
from jax.experimental.pallas import tpu as pltpu
from jax.experimental.pallas import tpu_sc as plsc

N = 50000
E = 800000
B = 64
D_IN = 25
H = 64
D_OUT = 128

NS = 16
CH = 112
NSL = 7
SW = 2
IR = NSL * SW
NWAVE = 32
NCH = IR * NWAVE
EPT = NCH * CH
E_PAD = EPT * NS
NWAVE1 = 16
NCH1 = IR * NWAVE1
ROWS = 50048
RPT = ROWS // NS
RPT_LAST = N - (NS - 1) * RPT

R = 2000
W1 = 32


def _sc_pipeline(xh, src, dst, zg, acc, sidx, didx, rows, gsem, ssem,
                 base0, nwave):
    def drain(j):
        pltpu.make_async_copy(zg, rows.at[j], ssem.at[j]).wait()

    def wave(w, carry):
        rowb = base0 + w * IR

        @pl.when(w > 0)
        def _():
            for j in range(NSL):
                drain(j)

        pltpu.sync_copy(src.at[pl.ds(rowb, IR)], sidx)
        pltpu.sync_copy(dst.at[pl.ds(rowb, IR)], didx)
        for sw in range(SW):
            if sw > 0:
                for j in range(NSL):
                    drain(j)
            gd = [pltpu.async_copy(xh.at[sidx.at[sw * NSL + j]],
                                   rows.at[j], gsem.at[j])
                  for j in range(NSL)]
            for j in range(NSL):
                gd[j].wait()
                pltpu.async_copy(rows.at[j], acc.at[didx.at[sw * NSL + j]],
                                 ssem.at[j], add=True)
        return carry

    lax.fori_loop(0, nwave, wave, 0)
    for j in range(NSL):
        drain(j)


def _copy_out(s, acc, out):
    r0 = s * RPT

    @pl.when(s < NS - 1)
    def _():
        pltpu.sync_copy(acc.at[pl.ds(r0, RPT)], out.at[pl.ds(r0, RPT)])

    @pl.when(s == NS - 1)
    def _():
        pltpu.sync_copy(acc.at[pl.ds(r0, RPT_LAST)],
                        out.at[pl.ds(r0, RPT_LAST)])


def _sc_scratch(Wh):
    return [
        pltpu.VMEM((IR, CH), jnp.int32),
        pltpu.VMEM((IR, CH), jnp.int32),
        pltpu.VMEM((NSL, CH, Wh), jnp.float32),
        pltpu.VMEM_SHARED((ROWS, Wh), jnp.float32),
        pltpu.SemaphoreType.DMA((NSL,)),
        pltpu.SemaphoreType.DMA((NSL,)),
    ]


@functools.cache
def _make_sc_agg_es():
    mesh = plsc.VectorSubcoreMesh(core_axis_name="c", subcore_axis_name="s")

    def body(x, src, dst, zrows, zg, out0, out1, sidx, didx, rows, acc,
             gsem, ssem):
        c = lax.axis_index("c")
        s = lax.axis_index("s")
        pltpu.sync_copy(zrows, acc.at[pl.ds(s * RPT, RPT)])
        plsc.subcore_barrier()
        base0 = (c * NS + s) * NCH1
        _sc_pipeline(x, src, dst, zg, acc, sidx, didx, rows, gsem, ssem,
                     base0, NWAVE1)
        plsc.subcore_barrier()

        @pl.when(c == 0)
        def _():
            _copy_out(s, acc, out0)

        @pl.when(c == 1)
        def _():
            _copy_out(s, acc, out1)

    return pl.kernel(
        body,
        out_type=(jax.ShapeDtypeStruct((N, W1), jnp.float32),
                  jax.ShapeDtypeStruct((N, W1), jnp.float32)),
        mesh=mesh,
        scratch_types=_sc_scratch(W1),
        compiler_params=pltpu.CompilerParams(use_tc_tiling_on_sc=False),
        name="sc_agg_es",
    )


@functools.cache
def _make_sc_agg_fs(Wh):
    mesh = plsc.VectorSubcoreMesh(core_axis_name="c", subcore_axis_name="s")

    def body(xlo, xhi, src, dst, zrows, zg, outlo, outhi, sidx, didx, rows,
             acc, gsem, ssem):
        c = lax.axis_index("c")
        s = lax.axis_index("s")
        pltpu.sync_copy(zrows, acc.at[pl.ds(s * RPT, RPT)])
        plsc.subcore_barrier()
        base0 = s * NCH

        def run(xh, out):
            _sc_pipeline(xh, src, dst, zg, acc, sidx, didx, rows, gsem,
                         ssem, base0, NWAVE)
            plsc.subcore_barrier()
            _copy_out(s, acc, out)

        @pl.when(c == 0)
        def _():
            run(xlo, outlo)

        @pl.when(c == 1)
        def _():
            run(xhi, outhi)

    return pl.kernel(
        body,
        out_type=(jax.ShapeDtypeStruct((N, Wh), jnp.float32),
                  jax.ShapeDtypeStruct((N, Wh), jnp.float32)),
        mesh=mesh,
        scratch_types=_sc_scratch(Wh),
        compiler_params=pltpu.CompilerParams(use_tc_tiling_on_sc=False),
        name=f"sc_agg_w{Wh}",
    )


def _pool_update(p, xr, bat, i):
    @pl.when(i == 0)
    def _():
        p[...] = jnp.full((B, H), -jnp.inf, jnp.float32)

    lo = bat[0, 0]
    hi = bat[R - 1, 0]

    def seg(bseg, carry):
        m = jnp.max(jnp.where(bat == bseg, xr, -jnp.inf),
                    axis=0, keepdims=True)
        p[pl.ds(bseg, 1), :] = jnp.maximum(p[pl.ds(bseg, 1), :], m)
        return carry

    lax.fori_loop(lo, hi + 1, seg, 0)


def _make_tc_layer1():
    grid = N // R

    def body(a0, a1, xf, bat, wl, wr, bias,
             xnlo, xnhi, p, invd_out):
        i = pl.program_id(0)
        a = a0[...] + a1[...]
        invd = 1.0 / jnp.maximum(a[:, D_IN:D_IN + 1], 1.0)
        invd_out[...] = invd
        h = ((a * invd) @ wl[...]
             + xf[...] @ wr[...]
             + bias[...])
        xr = jnp.maximum(h, 0.0)
        xnlo[...] = xr[:, :H // 2]
        xnhi[...] = xr[:, H // 2:]
        _pool_update(p, xr, bat[...], i)

    return pl.pallas_call(
        body,
        grid=(grid,),
        in_specs=[
            pl.BlockSpec((R, W1), lambda i: (i, 0)),
            pl.BlockSpec((R, W1), lambda i: (i, 0)),
            pl.BlockSpec((R, W1), lambda i: (i, 0)),
            pl.BlockSpec((R, 1), lambda i: (i, 0)),
            pl.BlockSpec((W1, H), lambda i: (0, 0)),
            pl.BlockSpec((W1, H), lambda i: (0, 0)),
            pl.BlockSpec((1, H), lambda i: (0, 0)),
        ],
        out_specs=[
            pl.BlockSpec((R, H // 2), lambda i: (i, 0)),
            pl.BlockSpec((R, H // 2), lambda i: (i, 0)),
            pl.BlockSpec((B, H), lambda i: (0, 0)),
            pl.BlockSpec((R, 1), lambda i: (i, 0)),
        ],
        out_shape=[
            jax.ShapeDtypeStruct((N, H // 2), jnp.float32),
            jax.ShapeDtypeStruct((N, H // 2), jnp.float32),
            jax.ShapeDtypeStruct((B, H), jnp.float32),
            jax.ShapeDtypeStruct((N, 1), jnp.float32),
        ],
        name="tc_layer1",
    )


def _make_tc_layer(last):
    grid = N // R
    Wa = H // 2

    def body(*refs):
        (alo, ahi, xlo, xhi, invd_r, bat, wllo, wlhi, wrlo, wrhi, bias,
         *rest) = refs
        if last:
            p1, p2, a1m, a2m, a3m, b1m, w2m, b2m, p, outm = rest
        else:
            xnlo, xnhi, p = rest
        i = pl.program_id(0)
        invd = invd_r[...]
        h = ((alo[...] * invd) @ wllo[...]
             + (ahi[...] * invd) @ wlhi[...]
             + xlo[...] @ wrlo[...]
             + xhi[...] @ wrhi[...]
             + bias[...])
        xr = jnp.maximum(h, 0.0)
        if not last:
            xnlo[...] = xr[:, :H // 2]
            xnhi[...] = xr[:, H // 2:]
        _pool_update(p, xr, bat[...], i)
        if last:
            @pl.when(i == grid - 1)
            def _():
                hm = jnp.maximum(p1[...] @ a1m[...] + p2[...] @ a2m[...]
                                 + p[...] @ a3m[...] + b1m[...], 0.0)
                outm[...] = hm @ w2m[...] + b2m[...]

    in_specs = [
        pl.BlockSpec((R, Wa), lambda i: (i, 0)),
        pl.BlockSpec((R, Wa), lambda i: (i, 0)),
        pl.BlockSpec((R, Wa), lambda i: (i, 0)),
        pl.BlockSpec((R, Wa), lambda i: (i, 0)),
        pl.BlockSpec((R, 1), lambda i: (i, 0)),
        pl.BlockSpec((R, 1), lambda i: (i, 0)),
        pl.BlockSpec((Wa, H), lambda i: (0, 0)),
        pl.BlockSpec((Wa, H), lambda i: (0, 0)),
        pl.BlockSpec((Wa, H), lambda i: (0, 0)),
        pl.BlockSpec((Wa, H), lambda i: (0, 0)),
        pl.BlockSpec((1, H), lambda i: (0, 0)),
    ]
    out_specs = [pl.BlockSpec((B, H), lambda i: (0, 0))]
    out_shape = [jax.ShapeDtypeStruct((B, H), jnp.float32)]
    if last:
        in_specs += [
            pl.BlockSpec((B, H), lambda i: (0, 0)),
            pl.BlockSpec((B, H), lambda i: (0, 0)),
            pl.BlockSpec((H, H), lambda i: (0, 0)),
            pl.BlockSpec((H, H), lambda i: (0, 0)),
            pl.BlockSpec((H, H), lambda i: (0, 0)),
            pl.BlockSpec((1, H), lambda i: (0, 0)),
            pl.BlockSpec((H, D_OUT), lambda i: (0, 0)),
            pl.BlockSpec((1, D_OUT), lambda i: (0, 0)),
        ]
        out_specs.append(pl.BlockSpec((B, D_OUT), lambda i: (0, 0)))
        out_shape.append(jax.ShapeDtypeStruct((B, D_OUT), jnp.float32))
    else:
        out_specs = [
            pl.BlockSpec((R, H // 2), lambda i: (i, 0)),
            pl.BlockSpec((R, H // 2), lambda i: (i, 0)),
        ] + out_specs
        out_shape = [
            jax.ShapeDtypeStruct((N, H // 2), jnp.float32),
            jax.ShapeDtypeStruct((N, H // 2), jnp.float32),
        ] + out_shape

    return pl.pallas_call(
        body,
        grid=(grid,),
        in_specs=in_specs,
        out_specs=out_specs,
        out_shape=out_shape,
        name="tc_layer3" if last else "tc_layer2",
    )


_tc_layer1 = _make_tc_layer1()
_tc_layer2 = _make_tc_layer(last=False)
_tc_layer3 = _make_tc_layer(last=True)


def kernel(x, edge_index, batch, W_l1, W_r1, b1, W_l2, W_r2, b2,
           W_l3, W_r3, b3, W_lin1, b_lin1, W_lin2, b_lin2):
    f32 = jnp.float32
    src = edge_index[0]
    dst = edge_index[1]
    pad = E_PAD - E
    src_p = jnp.concatenate([src, jnp.zeros((pad,), jnp.int32)])
    src_p = src_p.reshape(E_PAD // CH, CH)
    dst_p = jnp.concatenate([dst, jnp.full((pad,), N, jnp.int32)])
    dst_p = dst_p.reshape(E_PAD // CH, CH)

    x_pad = jnp.concatenate(
        [x, jnp.ones((N, 1), f32), jnp.zeros((N, W1 - D_IN - 1), f32)],
        axis=1)
    z32 = jnp.zeros((RPT, 32), f32)
    zg32 = jnp.zeros((CH, 32), f32)
    batch2 = batch[:, None]

    def pad_t(w):
        wt = w.T
        return jnp.concatenate([wt, jnp.zeros((W1 - D_IN, H), f32)], axis=0)

    wl1 = pad_t(W_l1)
    wr1 = pad_t(W_r1)
    wl2 = W_l2.T
    wr2 = W_r2.T
    wl3 = W_l3.T
    wr3 = W_r3.T
    w1t = W_lin1.T

    a10, a11 = _make_sc_agg_es()(x_pad, src_p, dst_p, z32, zg32)
    x1_lo, x1_hi, p1, invd = _tc_layer1(
        a10, a11, x_pad, batch2, wl1, wr1, b1[None])

    a2_lo, a2_hi = _make_sc_agg_fs(32)(x1_lo, x1_hi, src_p, dst_p, z32, zg32)
    x2_lo, x2_hi, p2 = _tc_layer2(
        a2_lo, a2_hi, x1_lo, x1_hi, invd, batch2,
        wl2[:32], wl2[32:], wr2[:32], wr2[32:], b2[None])

    a3_lo, a3_hi = _make_sc_agg_fs(32)(x2_lo, x2_hi, src_p, dst_p, z32, zg32)
    _, out = _tc_layer3(
        a3_lo, a3_hi, x2_lo, x2_hi, invd, batch2,
        wl3[:32], wl3[32:], wr3[:32], wr3[32:], b3[None],
        p1, p2, w1t[:H], w1t[H:2 * H], w1t[2 * H:],
        b_lin1[None], W_lin2.T, b_lin2[None])
    return out
